# baseline (speedup 1.0000x reference)
import functools

import jax
import jax.numpy as jnp
from jax import lax
from jax.experimental import pallas as pl
from jax.experimental.pallas import tpu as pltpu
from jax.experimental.pallas import tpu_sc as plsc

N_NODES = 10000
N_EDGES = 320000
D_FEAT = 128
D_HALF = D_FEAT // 2
BATCH_SIZE = 100
NODES_PER_GRAPH = 100

NC = 2
NS = 16
CHUNK = 128
EPT = 20096
NCHUNKS = EPT // CHUNK
E_PAD = NS * EPT
N_PAD = 10240
RPT = N_PAD // NS
ZCH = N_PAD // CHUNK // NS


def _sc_agg_body(with_deg, *refs):
    if with_deg:
        (h_hbm, eil_hbm, zfeat_hbm, zdeg_hbm, ones_hbm,
         agg_out, deg_out,
         agg_sh, deg_sh, rows0, rows1, rows2, rows3, eil_v, ones_v,
         dstage_v, sem0, sem1, sem2, sem3) = refs
    else:
        (h_hbm, eil_hbm, zfeat_hbm,
         agg_out,
         agg_sh, rows0, rows1, rows2, rows3, eil_v,
         sem0, sem1, sem2, sem3) = refs

    rows = (rows0, rows1, rows2, rows3)
    sems = (sem0, sem1, sem2, sem3)

    cid = lax.axis_index("c")
    sid = lax.axis_index("s")

    pltpu.sync_copy(eil_hbm.at[sid], eil_v)

    pltpu.sync_copy(zfeat_hbm, rows0)
    for k in range(ZCH):
        pltpu.sync_copy(
            rows0, agg_sh.at[pl.ds((sid * ZCH + k) * CHUNK, CHUNK)])
    if with_deg:
        pltpu.sync_copy(zdeg_hbm, dstage_v)
        pltpu.sync_copy(dstage_v, deg_sh.at[pl.ds(sid * RPT, RPT)])
        pltpu.sync_copy(ones_hbm, ones_v)
    plsc.subcore_barrier()

    h_c = h_hbm.at[cid]

    def gather(i, k):
        pltpu.async_copy(h_c.at[eil_v.at[i, 0]], rows[k], sems[k])

    def gwait(k):
        pltpu.make_async_copy(
            h_c.at[eil_v.at[0, 0]], rows[k], sems[k]).wait()

    def scatter(i, k):
        idx = eil_v.at[i, 1]
        pltpu.sync_copy(rows[k], agg_sh.at[idx], add=True)
        if with_deg:
            @pl.when(cid == 0)
            def _():
                pltpu.sync_copy(ones_v, deg_sh.at[idx], add=True)

    gather(0, 0)
    gather(1, 1)

    def group(g, carry):
        j0 = 4 * g
        for c in range(4):
            j = j0 + c
            gwait(c)
            gather(j + 2, (c + 2) % 4)
            scatter(j, c)
        return carry

    lax.fori_loop(0, (NCHUNKS - 5) // 4, group, 0)
    for j, c in ((NCHUNKS - 5, 0), (NCHUNKS - 4, 1), (NCHUNKS - 3, 2)):
        gwait(c)
        gather(j + 2, (c + 2) % 4)
        scatter(j, c)
    gwait(3)
    scatter(NCHUNKS - 2, 3)
    gwait(0)
    scatter(NCHUNKS - 1, 0)
    plsc.subcore_barrier()

    for k in range(ZCH):
        off = (sid * ZCH + k) * CHUNK
        pltpu.sync_copy(agg_sh.at[pl.ds(off, CHUNK)], rows0)
        pltpu.sync_copy(rows0, agg_out.at[cid, pl.ds(off, CHUNK)])
    if with_deg:
        @pl.when(cid == 0)
        def _():
            pltpu.sync_copy(deg_sh.at[pl.ds(sid * RPT, RPT)], dstage_v)
            pltpu.sync_copy(dstage_v, deg_out.at[pl.ds(sid * RPT, RPT)])


def _sc_aggregate(h_split, eil, with_deg):
    mesh = plsc.VectorSubcoreMesh(core_axis_name="c", subcore_axis_name="s",
                                  num_cores=NC, num_subcores=NS)
    zfeat = jnp.zeros((CHUNK, D_HALF), jnp.float32)
    rows_t = pltpu.VMEM((CHUNK, D_HALF), jnp.float32)
    if with_deg:
        out_type = (jax.ShapeDtypeStruct((NC, N_PAD, D_HALF), jnp.float32),
                    jax.ShapeDtypeStruct((N_PAD,), jnp.float32))
        scratch = [
            pltpu.VMEM_SHARED((N_PAD, D_HALF), jnp.float32),
            pltpu.VMEM_SHARED((N_PAD,), jnp.float32),
            rows_t, rows_t, rows_t, rows_t,
            pltpu.VMEM((NCHUNKS, 2, CHUNK), jnp.int32),
            pltpu.VMEM((CHUNK,), jnp.float32),
            pltpu.VMEM((RPT,), jnp.float32),
            pltpu.SemaphoreType.DMA,
            pltpu.SemaphoreType.DMA,
            pltpu.SemaphoreType.DMA,
            pltpu.SemaphoreType.DMA,
        ]
        zdeg = jnp.zeros((RPT,), jnp.float32)
        ones = jnp.ones((CHUNK,), jnp.float32)
        fn = pl.kernel(functools.partial(_sc_agg_body, True),
                       out_type=out_type, mesh=mesh, scratch_types=scratch,
                       compiler_params=pltpu.CompilerParams(
                           use_tc_tiling_on_sc=False))
        return fn(h_split, eil, zfeat, zdeg, ones)
    else:
        out_type = jax.ShapeDtypeStruct((NC, N_PAD, D_HALF), jnp.float32)
        scratch = [
            pltpu.VMEM_SHARED((N_PAD, D_HALF), jnp.float32),
            rows_t, rows_t, rows_t, rows_t,
            pltpu.VMEM((NCHUNKS, 2, CHUNK), jnp.int32),
            pltpu.SemaphoreType.DMA,
            pltpu.SemaphoreType.DMA,
            pltpu.SemaphoreType.DMA,
            pltpu.SemaphoreType.DMA,
        ]
        fn = pl.kernel(functools.partial(_sc_agg_body, False),
                       out_type=out_type, mesh=mesh, scratch_types=scratch,
                       compiler_params=pltpu.CompilerParams(
                           use_tc_tiling_on_sc=False))
        return fn(h_split, eil, zfeat)


def _dense1_body(x_ref, agg_ref, deg_ref, Wr_ref, Wa_ref, b_ref, g_ref,
                 be_ref, o_ref):
    agg = jnp.concatenate(
        [agg_ref[0, :N_NODES], agg_ref[1, :N_NODES]], axis=1)
    deg = deg_ref[:N_NODES]
    mean = agg / jnp.maximum(deg, 1.0)
    c = (jnp.dot(x_ref[...], Wr_ref[...], preferred_element_type=jnp.float32)
         + jnp.dot(mean, Wa_ref[...], preferred_element_type=jnp.float32)
         + b_ref[...])
    mu = jnp.mean(c, axis=0, keepdims=True)
    var = jnp.mean((c - mu) * (c - mu), axis=0, keepdims=True)
    h = (c - mu) * lax.rsqrt(var + 1e-5) * g_ref[...] + be_ref[...]
    h = jnp.maximum(h, 0.0)
    o_ref[0] = h[:, :D_HALF]
    o_ref[1] = h[:, D_HALF:]


def _dense1(x, agg, deg, Wr, Wa, b, g, be):
    return pl.pallas_call(
        _dense1_body,
        out_shape=jax.ShapeDtypeStruct((NC, N_NODES, D_HALF), jnp.float32),
    )(x, agg, deg, Wr, Wa, b.reshape(1, -1),
      g.reshape(1, -1), be.reshape(1, -1))


def _dense2_body(h_ref, agg_ref, deg_ref, Wr_ref, Wa_ref, b_ref, g_ref,
                 be_ref, Ws1_ref, bs1_ref, Ws2_ref, bs2_ref, Wh1_ref,
                 bh1_ref, Wh2_ref, bh2_ref, Wh3_ref, bh3_ref, o_ref):
    hin = jnp.concatenate([h_ref[0], h_ref[1]], axis=1)
    agg = jnp.concatenate(
        [agg_ref[0, :N_NODES], agg_ref[1, :N_NODES]], axis=1)
    deg = deg_ref[:N_NODES]
    mean = agg / jnp.maximum(deg, 1.0)
    c = (jnp.dot(hin, Wr_ref[...], preferred_element_type=jnp.float32)
         + jnp.dot(mean, Wa_ref[...], preferred_element_type=jnp.float32)
         + b_ref[...])
    mu = jnp.mean(c, axis=0, keepdims=True)
    var = jnp.mean((c - mu) * (c - mu), axis=0, keepdims=True)
    h = (c - mu) * lax.rsqrt(var + 1e-5) * g_ref[...] + be_ref[...]
    h = jnp.maximum(h, 0.0)

    hg = jnp.mean(h.reshape(BATCH_SIZE, NODES_PER_GRAPH, D_FEAT), axis=1)

    t = jnp.maximum(hg, 0.0)
    t = jnp.dot(t, Ws1_ref[...], preferred_element_type=jnp.float32) + bs1_ref[...]
    t = jnp.dot(t, Ws2_ref[...], preferred_element_type=jnp.float32) + bs2_ref[...]
    t = jnp.maximum(t, 0.0)
    t = jnp.maximum(jnp.dot(t, Wh1_ref[...], preferred_element_type=jnp.float32) + bh1_ref[...], 0.0)
    t = jnp.maximum(jnp.dot(t, Wh2_ref[...], preferred_element_type=jnp.float32) + bh2_ref[...], 0.0)
    o_ref[...] = jnp.dot(t, Wh3_ref[...], preferred_element_type=jnp.float32) + bh3_ref[...]


def _dense2(h_split, agg, deg, Wr, Wa, b, g, be, Ws1, bs1, Ws2, bs2,
            Wh1, bh1, Wh2, bh2, Wh3, bh3):
    return pl.pallas_call(
        _dense2_body,
        out_shape=jax.ShapeDtypeStruct((BATCH_SIZE, 1), jnp.float32),
    )(h_split, agg, deg, Wr, Wa, b.reshape(1, -1), g.reshape(1, -1),
      be.reshape(1, -1), Ws1, bs1.reshape(1, -1), Ws2, bs2.reshape(1, -1),
      Wh1, bh1.reshape(1, -1), Wh2, bh2.reshape(1, -1), Wh3,
      bh3.reshape(1, -1))


def kernel(x, edge_index, batch, Wr0, Wa0, b0, g0, be0, Wr1, Wa1, b1, g1,
           be1, Ws1, bs1, Ws2, bs2, Wh1, bh1, Wh2, bh2, Wh3, bh3):
    src = edge_index[0]
    dst = edge_index[1]
    npad = E_PAD - N_EDGES
    src_p = jnp.concatenate([src, jnp.zeros((npad,), jnp.int32)])
    dst_p = jnp.concatenate([dst, jnp.full((npad,), N_PAD - 1, jnp.int32)])
    eil = jnp.stack([src_p.reshape(NS, NCHUNKS, CHUNK),
                     dst_p.reshape(NS, NCHUNKS, CHUNK)], axis=2)
    x_split = jnp.stack([x[:, :D_HALF], x[:, D_HALF:]])

    agg0, deg_flat = _sc_aggregate(x_split, eil, with_deg=True)
    deg = deg_flat.reshape(N_PAD, 1)
    h1_split = _dense1(x, agg0, deg, Wr0, Wa0, b0, g0, be0)
    agg1 = _sc_aggregate(h1_split, eil, with_deg=False)
    return _dense2(h1_split, agg1, deg, Wr1, Wa1, b1, g1, be1, Ws1, bs1,
                   Ws2, bs2, Wh1, bh1, Wh2, bh2, Wh3, bh3)
